# baseline (speedup 1.0000x reference)
import functools

import jax
import jax.numpy as jnp
from jax import lax
from jax.experimental import pallas as pl
from jax.experimental.pallas import tpu as pltpu
from jax.experimental.pallas import tpu_sc as plsc

N_VARS = 50000
NPAD = 50048
E_EDGES = 1600000
NW = 32
SEG_PER_TILE = 1568
SP = SEG_PER_TILE * NW
CHUNK = 4096
EDGE_PAD = 2 * CHUNK + 16
NB = 48

_MESH = plsc.VectorSubcoreMesh(
    core_axis_name="c", subcore_axis_name="s", num_cores=2, num_subcores=16
)


_GDN = lax.GatherDimensionNumbers(
    offset_dims=(), collapsed_slice_dims=(0,), start_index_map=(0,)
)


def _vgather(x, idx):
    return lax.gather(x, idx[:, None], _GDN, slice_sizes=(1,),
                      mode=lax.GatherScatterMode.PROMISE_IN_BOUNDS)


def _seg_reduce_body(tab, ixin, ixout, bounds, out, tab_v, acc, bi, bo, bnd_v,
                     *, transform, apply_exp):
    wid = lax.axis_index("s") * 2 + lax.axis_index("c")
    pltpu.sync_copy(tab, tab_v)
    pltpu.sync_copy(bounds, bnd_v)
    seg_base = pl.multiple_of(wid * SEG_PER_TILE, 16)
    e_lo = bnd_v[pl.ds(wid, 16)][0]
    e_hi = bnd_v[pl.ds(wid + 1, 16)][0]

    zeros16 = jnp.zeros((16,), jnp.float32)

    def zero_body(i, _):
        acc[pl.ds(i * 16, 16)] = zeros16
        return 0

    lax.fori_loop(0, SEG_PER_TILE // 16, zero_body, 0)

    iot = lax.iota(jnp.int32, 16)
    prev_l = jnp.maximum(iot - 1, 0)
    next_l = jnp.minimum(iot + 1, 15)

    base = e_lo & ~15
    nch = (e_hi - base + CHUNK - 1) // CHUNK

    def chunk_body(k, _):
        off = pl.multiple_of(base + k * CHUNK, 16)
        pltpu.sync_copy(ixin.at[pl.ds(off, CHUNK)], bi)
        pltpu.sync_copy(ixout.at[pl.ds(off, CHUNK)], bo)

        @plsc.parallel_loop(0, CHUNK, step=16, unroll=8)
        def vreg_body(j):
            io = bo[pl.ds(j, 16)]
            ii = bi[pl.ds(j, 16)]
            v = plsc.load_gather(tab_v, [transform(ii)])
            incl = plsc.cumsum(v)
            excl = incl - v
            pio = _vgather(io, prev_l)
            nio = _vgather(io, next_l)
            start = (iot == 0) | (io != pio)
            last = (iot == 15) | (io != nio)
            rs = plsc.cummax(jnp.where(start, iot, 0))
            part = incl - _vgather(excl, rs)
            eid = off + j + iot
            m = last & (eid >= e_lo) & (eid < e_hi)
            lidx = jnp.clip(io - seg_base, 0, SEG_PER_TILE - 1)
            plsc.addupdate_scatter(acc, [lidx], part, mask=m)

        return 0

    lax.fori_loop(0, nch, chunk_body, 0)

    if apply_exp:
        def exp_body(i, _):
            acc[pl.ds(i * 16, 16)] = jnp.exp(acc[pl.ds(i * 16, 16)])
            return 0

        lax.fori_loop(0, SEG_PER_TILE // 16, exp_body, 0)

    pltpu.sync_copy(acc, out.at[pl.ds(seg_base, SEG_PER_TILE)])


def _transform_layer0(ii):
    j2 = ii - 2
    return (j2 >> 1) + (j2 & 1) * NPAD


def _make_seg_kernel(tab_len, transform, apply_exp):
    scratch = [
        pltpu.VMEM((tab_len,), jnp.float32),
        pltpu.VMEM((SEG_PER_TILE,), jnp.float32),
        pltpu.VMEM((CHUNK,), jnp.int32),
        pltpu.VMEM((CHUNK,), jnp.int32),
        pltpu.VMEM((NB,), jnp.int32),
    ]

    @functools.partial(
        pl.kernel,
        out_type=jax.ShapeDtypeStruct((SP,), jnp.float32),
        mesh=_MESH,
        scratch_types=scratch,
        compiler_params=pltpu.CompilerParams(needs_layout_passes=False),
    )
    def k(tab, ixin, ixout, bounds, out, tab_v, acc, bi, bo, bnd_v):
        _seg_reduce_body(tab, ixin, ixout, bounds, out, tab_v, acc, bi, bo,
                         bnd_v, transform=transform, apply_exp=apply_exp)

    return k


_layer0 = _make_seg_kernel(2 * NPAD, _transform_layer0, True)
_layer1 = _make_seg_kernel(SP, lambda ii: ii, False)


def _log_table(x_pos):
    xp = jnp.pad(x_pos, (0, NPAD - N_VARS), constant_values=0.5)
    xp = xp.reshape(NPAD // 128, 128)

    def body(x_ref, lp_ref, ln_ref):
        x = x_ref[...]
        lp_ref[...] = jnp.log(x)
        ln_ref[...] = jnp.log(1.0 - x)

    lp, ln = pl.pallas_call(
        body,
        out_shape=[jax.ShapeDtypeStruct((NPAD // 128, 128), jnp.float32)] * 2,
    )(xp)
    return jnp.concatenate([lp.reshape(-1), ln.reshape(-1)])


def kernel(x_pos, ix_in0, ix_out0, ix_in1, ix_out1):
    ix_in0 = ix_in0.astype(jnp.int32)
    ix_out0 = ix_out0.astype(jnp.int32)
    ix_in1 = ix_in1.astype(jnp.int32)
    ix_out1 = ix_out1.astype(jnp.int32)

    ltab = _log_table(x_pos)

    seg_starts = jnp.arange(NW + 1, dtype=jnp.int32) * SEG_PER_TILE
    b0 = jnp.pad(jnp.searchsorted(ix_out0, seg_starts).astype(jnp.int32),
                 (0, NB - (NW + 1)))
    b1 = jnp.pad(jnp.searchsorted(ix_out1, seg_starts).astype(jnp.int32),
                 (0, NB - (NW + 1)))

    ixin0 = jnp.pad(ix_in0, (0, EDGE_PAD), constant_values=2)
    ixout0 = jnp.pad(ix_out0, (0, EDGE_PAD), constant_values=SP)
    ixin1 = jnp.pad(ix_in1, (0, EDGE_PAD), constant_values=0)
    ixout1 = jnp.pad(ix_out1, (0, EDGE_PAD), constant_values=SP)

    h0 = _layer0(ltab, ixin0, ixout0, b0)
    h1 = _layer1(h0, ixin1, ixout1, b1)
    return h1[:N_VARS]


# speedup vs baseline: 1.0163x; 1.0163x over previous
import functools

import jax
import jax.numpy as jnp
from jax import lax
from jax.experimental import pallas as pl
from jax.experimental.pallas import tpu as pltpu
from jax.experimental.pallas import tpu_sc as plsc

N_VARS = 50000
NPAD = 50048
E_EDGES = 1600000
NW = 32
SEG_PER_TILE = 1568
SP = SEG_PER_TILE * NW
CHUNK = 4096
EDGE_PAD = 2 * CHUNK + 16
NB = 48

_MESH = plsc.VectorSubcoreMesh(
    core_axis_name="c", subcore_axis_name="s", num_cores=2, num_subcores=16
)


_GDN = lax.GatherDimensionNumbers(
    offset_dims=(), collapsed_slice_dims=(0,), start_index_map=(0,)
)


def _vgather(x, idx):
    return lax.gather(x, idx[:, None], _GDN, slice_sizes=(1,),
                      mode=lax.GatherScatterMode.PROMISE_IN_BOUNDS)


def _seg_reduce_body(tab, ixin, ixout, bounds, out, tab_v, acc, bi, bo, bnd_v,
                     *, transform, apply_exp):
    wid = lax.axis_index("s") * 2 + lax.axis_index("c")
    pltpu.sync_copy(tab, tab_v)
    pltpu.sync_copy(bounds, bnd_v)
    seg_base = pl.multiple_of(wid * SEG_PER_TILE, 16)
    e_lo = bnd_v[pl.ds(wid, 16)][0]
    e_hi = bnd_v[pl.ds(wid + 1, 16)][0]

    zeros16 = jnp.zeros((16,), jnp.float32)

    def zero_body(i, _):
        acc[pl.ds(i * 16, 16)] = zeros16
        return 0

    lax.fori_loop(0, SEG_PER_TILE // 16, zero_body, 0)

    iot = lax.iota(jnp.int32, 16)
    prev_l = jnp.maximum(iot - 1, 0)
    next_l = jnp.minimum(iot + 1, 15)

    base = e_lo & ~15
    nch = (e_hi - base + CHUNK - 1) // CHUNK

    def chunk_body(k, _):
        off = pl.multiple_of(base + k * CHUNK, 16)
        pltpu.sync_copy(ixin.at[pl.ds(off, CHUNK)], bi)
        pltpu.sync_copy(ixout.at[pl.ds(off, CHUNK)], bo)

        @plsc.parallel_loop(0, CHUNK, step=16, unroll=4)
        def vreg_body(j):
            io = bo[pl.ds(j, 16)]
            ii = bi[pl.ds(j, 16)]
            v = plsc.load_gather(tab_v, [transform(ii)])
            incl = plsc.cumsum(v)
            excl = incl - v
            pio = _vgather(io, prev_l)
            nio = _vgather(io, next_l)
            start = (iot == 0) | (io != pio)
            last = (iot == 15) | (io != nio)
            rs = plsc.cummax(jnp.where(start, iot, 0))
            part = incl - _vgather(excl, rs)
            eid = off + j + iot
            m = last & (eid >= e_lo) & (eid < e_hi)
            lidx = jnp.clip(io - seg_base, 0, SEG_PER_TILE - 1)
            plsc.addupdate_scatter(acc, [lidx], part, mask=m)

        return 0

    lax.fori_loop(0, nch, chunk_body, 0)

    if apply_exp:
        def exp_body(i, _):
            acc[pl.ds(i * 16, 16)] = jnp.exp(acc[pl.ds(i * 16, 16)])
            return 0

        lax.fori_loop(0, SEG_PER_TILE // 16, exp_body, 0)

    pltpu.sync_copy(acc, out.at[pl.ds(seg_base, SEG_PER_TILE)])


def _transform_layer0(ii):
    j2 = ii - 2
    return (j2 >> 1) + (j2 & 1) * NPAD


def _make_seg_kernel(tab_len, transform, apply_exp):
    scratch = [
        pltpu.VMEM((tab_len,), jnp.float32),
        pltpu.VMEM((SEG_PER_TILE,), jnp.float32),
        pltpu.VMEM((CHUNK,), jnp.int32),
        pltpu.VMEM((CHUNK,), jnp.int32),
        pltpu.VMEM((NB,), jnp.int32),
    ]

    @functools.partial(
        pl.kernel,
        out_type=jax.ShapeDtypeStruct((SP,), jnp.float32),
        mesh=_MESH,
        scratch_types=scratch,
        compiler_params=pltpu.CompilerParams(needs_layout_passes=False),
    )
    def k(tab, ixin, ixout, bounds, out, tab_v, acc, bi, bo, bnd_v):
        _seg_reduce_body(tab, ixin, ixout, bounds, out, tab_v, acc, bi, bo,
                         bnd_v, transform=transform, apply_exp=apply_exp)

    return k


_layer0 = _make_seg_kernel(2 * NPAD, _transform_layer0, True)
_layer1 = _make_seg_kernel(SP, lambda ii: ii, False)


def _log_table(x_pos):
    xp = jnp.pad(x_pos, (0, NPAD - N_VARS), constant_values=0.5)
    xp = xp.reshape(NPAD // 128, 128)

    def body(x_ref, lp_ref, ln_ref):
        x = x_ref[...]
        lp_ref[...] = jnp.log(x)
        ln_ref[...] = jnp.log(1.0 - x)

    lp, ln = pl.pallas_call(
        body,
        out_shape=[jax.ShapeDtypeStruct((NPAD // 128, 128), jnp.float32)] * 2,
    )(xp)
    return jnp.concatenate([lp.reshape(-1), ln.reshape(-1)])


def kernel(x_pos, ix_in0, ix_out0, ix_in1, ix_out1):
    ix_in0 = ix_in0.astype(jnp.int32)
    ix_out0 = ix_out0.astype(jnp.int32)
    ix_in1 = ix_in1.astype(jnp.int32)
    ix_out1 = ix_out1.astype(jnp.int32)

    ltab = _log_table(x_pos)

    seg_starts = jnp.arange(NW + 1, dtype=jnp.int32) * SEG_PER_TILE
    b0 = jnp.pad(jnp.searchsorted(ix_out0, seg_starts).astype(jnp.int32),
                 (0, NB - (NW + 1)))
    b1 = jnp.pad(jnp.searchsorted(ix_out1, seg_starts).astype(jnp.int32),
                 (0, NB - (NW + 1)))

    ixin0 = jnp.pad(ix_in0, (0, EDGE_PAD), constant_values=2)
    ixout0 = jnp.pad(ix_out0, (0, EDGE_PAD), constant_values=SP)
    ixin1 = jnp.pad(ix_in1, (0, EDGE_PAD), constant_values=0)
    ixout1 = jnp.pad(ix_out1, (0, EDGE_PAD), constant_values=SP)

    h0 = _layer0(ltab, ixin0, ixout0, b0)
    h1 = _layer1(h0, ixin1, ixout1, b1)
    return h1[:N_VARS]


# speedup vs baseline: 1.1509x; 1.1325x over previous
import functools

import jax
import jax.numpy as jnp
from jax import lax
from jax.experimental import pallas as pl
from jax.experimental.pallas import tpu as pltpu
from jax.experimental.pallas import tpu_sc as plsc

N_VARS = 50000
NPAD = 50048
E_EDGES = 1600000
NW = 32
SEG_PER_TILE = 1568
SP = SEG_PER_TILE * NW
CHUNK = 4096
EDGE_PAD = 2 * CHUNK + 16
NB = 48

_MESH = plsc.VectorSubcoreMesh(
    core_axis_name="c", subcore_axis_name="s", num_cores=2, num_subcores=16
)


_GDN = lax.GatherDimensionNumbers(
    offset_dims=(), collapsed_slice_dims=(0,), start_index_map=(0,)
)


def _vgather(x, idx):
    return lax.gather(x, idx[:, None], _GDN, slice_sizes=(1,),
                      mode=lax.GatherScatterMode.PROMISE_IN_BOUNDS)


def _seg_reduce_body(tab, ixin, ixout, bounds, out, tab_v, acc, bi, bo, bnd_v,
                     *, transform, apply_exp):
    wid = lax.axis_index("s") * 2 + lax.axis_index("c")
    pltpu.sync_copy(tab, tab_v)
    pltpu.sync_copy(bounds, bnd_v)
    seg_base = pl.multiple_of(wid * SEG_PER_TILE, 16)
    e_lo = bnd_v[pl.ds(wid, 16)][0]
    e_hi = bnd_v[pl.ds(wid + 1, 16)][0]

    zeros16 = jnp.zeros((16,), jnp.float32)

    def zero_body(i, _):
        acc[pl.ds(i * 16, 16)] = zeros16
        return 0

    lax.fori_loop(0, SEG_PER_TILE // 16, zero_body, 0)

    iot = lax.iota(jnp.int32, 16)
    prev_l = jnp.maximum(iot - 1, 0)
    next_l = jnp.minimum(iot + 1, 15)

    base = e_lo & ~15
    nch = (e_hi - base + CHUNK - 1) // CHUNK

    def chunk_body(k, _):
        pos = base + k * CHUNK
        off = pl.multiple_of(jnp.minimum(pos, E_EDGES - CHUNK), 16)
        lo_k = jnp.maximum(e_lo, pos)
        pltpu.sync_copy(ixin.at[pl.ds(off, CHUNK)], bi)
        pltpu.sync_copy(ixout.at[pl.ds(off, CHUNK)], bo)

        @plsc.parallel_loop(0, CHUNK, step=16, unroll=4)
        def vreg_body(j):
            io = bo[pl.ds(j, 16)]
            ii = bi[pl.ds(j, 16)]
            v = plsc.load_gather(tab_v, [transform(ii)])
            incl = plsc.cumsum(v)
            excl = incl - v
            pio = _vgather(io, prev_l)
            nio = _vgather(io, next_l)
            start = (iot == 0) | (io != pio)
            last = (iot == 15) | (io != nio)
            rs = plsc.cummax(jnp.where(start, iot, 0))
            part = incl - _vgather(excl, rs)
            eid = off + j + iot
            m = last & (eid >= lo_k) & (eid < e_hi)
            lidx = jnp.clip(io - seg_base, 0, SEG_PER_TILE - 1)
            plsc.addupdate_scatter(acc, [lidx], part, mask=m)

        return 0

    lax.fori_loop(0, nch, chunk_body, 0)

    if apply_exp:
        def exp_body(i, _):
            acc[pl.ds(i * 16, 16)] = jnp.exp(acc[pl.ds(i * 16, 16)])
            return 0

        lax.fori_loop(0, SEG_PER_TILE // 16, exp_body, 0)

    pltpu.sync_copy(acc, out.at[pl.ds(seg_base, SEG_PER_TILE)])


def _transform_layer0(ii):
    j2 = ii - 2
    return (j2 >> 1) + (j2 & 1) * NPAD


def _make_seg_kernel(tab_len, transform, apply_exp):
    scratch = [
        pltpu.VMEM((tab_len,), jnp.float32),
        pltpu.VMEM((SEG_PER_TILE,), jnp.float32),
        pltpu.VMEM((CHUNK,), jnp.int32),
        pltpu.VMEM((CHUNK,), jnp.int32),
        pltpu.VMEM((NB,), jnp.int32),
    ]

    @functools.partial(
        pl.kernel,
        out_type=jax.ShapeDtypeStruct((SP,), jnp.float32),
        mesh=_MESH,
        scratch_types=scratch,
        compiler_params=pltpu.CompilerParams(needs_layout_passes=False),
    )
    def k(tab, ixin, ixout, bounds, out, tab_v, acc, bi, bo, bnd_v):
        _seg_reduce_body(tab, ixin, ixout, bounds, out, tab_v, acc, bi, bo,
                         bnd_v, transform=transform, apply_exp=apply_exp)

    return k


_layer0 = _make_seg_kernel(2 * NPAD, _transform_layer0, True)
_layer1 = _make_seg_kernel(SP, lambda ii: ii, False)


def _log_table(x_pos):
    xp = jnp.pad(x_pos, (0, NPAD - N_VARS), constant_values=0.5)
    xp = xp.reshape(NPAD // 128, 128)

    def body(x_ref, o_ref):
        x = x_ref[...]
        o_ref[0] = jnp.log(x)
        o_ref[1] = jnp.log(1.0 - x)

    out = pl.pallas_call(
        body,
        out_shape=jax.ShapeDtypeStruct((2, NPAD // 128, 128), jnp.float32),
    )(xp)
    return out.reshape(-1)


def kernel(x_pos, ix_in0, ix_out0, ix_in1, ix_out1):
    ix_in0 = ix_in0.astype(jnp.int32)
    ix_out0 = ix_out0.astype(jnp.int32)
    ix_in1 = ix_in1.astype(jnp.int32)
    ix_out1 = ix_out1.astype(jnp.int32)

    ltab = _log_table(x_pos)

    seg_starts = jnp.arange(NW + 1, dtype=jnp.int32) * SEG_PER_TILE
    b0 = jnp.pad(jnp.searchsorted(ix_out0, seg_starts).astype(jnp.int32),
                 (0, NB - (NW + 1)))
    b1 = jnp.pad(jnp.searchsorted(ix_out1, seg_starts).astype(jnp.int32),
                 (0, NB - (NW + 1)))

    h0 = _layer0(ltab, ix_in0, ix_out0, b0)
    h1 = _layer1(h0, ix_in1, ix_out1, b1)
    return h1[:N_VARS]


# speedup vs baseline: 1.3036x; 1.1326x over previous
import functools

import jax
import jax.numpy as jnp
from jax import lax
from jax.experimental import pallas as pl
from jax.experimental.pallas import tpu as pltpu
from jax.experimental.pallas import tpu_sc as plsc

N_VARS = 50000
NPAD = 50048
E_EDGES = 1600000
NW = 32
SEG_PER_TILE = 1568
SP = SEG_PER_TILE * NW
CHUNK = 4096
EDGE_PAD = 2 * CHUNK + 16
NB = 48

_MESH = plsc.VectorSubcoreMesh(
    core_axis_name="c", subcore_axis_name="s", num_cores=2, num_subcores=16
)


_GDN = lax.GatherDimensionNumbers(
    offset_dims=(), collapsed_slice_dims=(0,), start_index_map=(0,)
)


def _vgather(x, idx):
    return lax.gather(x, idx[:, None], _GDN, slice_sizes=(1,),
                      mode=lax.GatherScatterMode.PROMISE_IN_BOUNDS)


def _seg_reduce_body(tab, ixin, ixout, bounds, out, tab_v, acc,
                     bi0, bo0, bi1, bo1, bnd_v, sem_t, sem0, sem1,
                     *, transform, apply_exp):
    wid = lax.axis_index("s") * 2 + lax.axis_index("c")
    tcopy = pltpu.async_copy(tab, tab_v, sem_t)
    pltpu.sync_copy(bounds, bnd_v)
    seg_base = pl.multiple_of(wid * SEG_PER_TILE, 16)
    e_lo = bnd_v[pl.ds(wid, 16)][0]
    e_hi = bnd_v[pl.ds(wid + 1, 16)][0]

    zeros16 = jnp.zeros((16,), jnp.float32)

    def zero_body(i, _):
        acc[pl.ds(i * 16, 16)] = zeros16
        return 0

    lax.fori_loop(0, SEG_PER_TILE // 16, zero_body, 0)

    iot = lax.iota(jnp.int32, 16)
    prev_l = jnp.maximum(iot - 1, 0)
    next_l = jnp.minimum(iot + 1, 15)

    base = e_lo & ~15
    nch = (e_hi - base + CHUNK - 1) // CHUNK

    def chunk_off(k):
        pos = base + k * CHUNK
        return pl.multiple_of(jnp.minimum(pos, E_EDGES - CHUNK), 16)

    def start_dma(k, bi_, bo_, sem_):
        off = chunk_off(k)
        pltpu.async_copy(ixin.at[pl.ds(off, CHUNK)], bi_, sem_)
        pltpu.async_copy(ixout.at[pl.ds(off, CHUNK)], bo_, sem_)

    def wait_dma(bi_, bo_, sem_):
        pltpu.make_async_copy(ixin.at[pl.ds(0, CHUNK)], bi_, sem_).wait()
        pltpu.make_async_copy(ixout.at[pl.ds(0, CHUNK)], bo_, sem_).wait()

    @pl.when(nch > 0)
    def _():
        start_dma(0, bi0, bo0, sem0)

    tcopy.wait()

    def compute_chunk(k, bi_, bo_):
        off = chunk_off(k)
        lo_k = jnp.maximum(e_lo, base + k * CHUNK)

        @plsc.parallel_loop(0, CHUNK, step=16, unroll=4)
        def vreg_body(j):
            io = bo_[pl.ds(j, 16)]
            ii = bi_[pl.ds(j, 16)]
            v = plsc.load_gather(tab_v, [transform(ii)])
            incl = plsc.cumsum(v)
            excl = incl - v
            pio = _vgather(io, prev_l)
            nio = _vgather(io, next_l)
            start = (iot == 0) | (io != pio)
            last = (iot == 15) | (io != nio)
            rs = plsc.cummax(jnp.where(start, iot, 0))
            part = incl - _vgather(excl, rs)
            eid = off + j + iot
            m = last & (eid >= lo_k) & (eid < e_hi)
            plsc.addupdate_scatter(acc, [io - seg_base], part, mask=m)

    def chunk_body(k, _):
        @pl.when(k % 2 == 0)
        def _():
            wait_dma(bi0, bo0, sem0)

            @pl.when(k + 1 < nch)
            def _():
                start_dma(k + 1, bi1, bo1, sem1)

            compute_chunk(k, bi0, bo0)

        @pl.when(k % 2 == 1)
        def _():
            wait_dma(bi1, bo1, sem1)

            @pl.when(k + 1 < nch)
            def _():
                start_dma(k + 1, bi0, bo0, sem0)

            compute_chunk(k, bi1, bo1)

        return 0

    lax.fori_loop(0, nch, chunk_body, 0)

    if apply_exp:
        def exp_body(i, _):
            acc[pl.ds(i * 16, 16)] = jnp.exp(acc[pl.ds(i * 16, 16)])
            return 0

        lax.fori_loop(0, SEG_PER_TILE // 16, exp_body, 0)

    pltpu.sync_copy(acc, out.at[pl.ds(seg_base, SEG_PER_TILE)])


def _transform_layer0(ii):
    j2 = ii - 2
    return (j2 >> 1) + (j2 & 1) * NPAD


def _make_seg_kernel(tab_len, transform, apply_exp):
    scratch = [
        pltpu.VMEM((tab_len,), jnp.float32),
        pltpu.VMEM((SEG_PER_TILE,), jnp.float32),
        pltpu.VMEM((CHUNK,), jnp.int32),
        pltpu.VMEM((CHUNK,), jnp.int32),
        pltpu.VMEM((CHUNK,), jnp.int32),
        pltpu.VMEM((CHUNK,), jnp.int32),
        pltpu.VMEM((NB,), jnp.int32),
        pltpu.SemaphoreType.DMA,
        pltpu.SemaphoreType.DMA,
        pltpu.SemaphoreType.DMA,
    ]

    @functools.partial(
        pl.kernel,
        out_type=jax.ShapeDtypeStruct((SP,), jnp.float32),
        mesh=_MESH,
        scratch_types=scratch,
        compiler_params=pltpu.CompilerParams(needs_layout_passes=False),
    )
    def k(tab, ixin, ixout, bounds, out, tab_v, acc, bi0, bo0, bi1, bo1,
          bnd_v, sem_t, sem0, sem1):
        _seg_reduce_body(tab, ixin, ixout, bounds, out, tab_v, acc,
                         bi0, bo0, bi1, bo1, bnd_v, sem_t, sem0, sem1,
                         transform=transform, apply_exp=apply_exp)

    return k


_layer0 = _make_seg_kernel(2 * NPAD, _transform_layer0, True)
_layer1 = _make_seg_kernel(SP, lambda ii: ii, False)


def _log_table(x_pos):
    xp = jnp.pad(x_pos, (0, NPAD - N_VARS), constant_values=0.5)
    xp = xp.reshape(NPAD // 128, 128)

    def body(x_ref, o_ref):
        x = x_ref[...]
        o_ref[0] = jnp.log(x)
        o_ref[1] = jnp.log(1.0 - x)

    out = pl.pallas_call(
        body,
        out_shape=jax.ShapeDtypeStruct((2, NPAD // 128, 128), jnp.float32),
    )(xp)
    return out.reshape(-1)


def kernel(x_pos, ix_in0, ix_out0, ix_in1, ix_out1):
    ix_in0 = ix_in0.astype(jnp.int32)
    ix_out0 = ix_out0.astype(jnp.int32)
    ix_in1 = ix_in1.astype(jnp.int32)
    ix_out1 = ix_out1.astype(jnp.int32)

    ltab = _log_table(x_pos)

    seg_starts = jnp.arange(NW + 1, dtype=jnp.int32) * SEG_PER_TILE
    b0 = jnp.pad(jnp.searchsorted(ix_out0, seg_starts).astype(jnp.int32),
                 (0, NB - (NW + 1)))
    b1 = jnp.pad(jnp.searchsorted(ix_out1, seg_starts).astype(jnp.int32),
                 (0, NB - (NW + 1)))

    h0 = _layer0(ltab, ix_in0, ix_out0, b0)
    h1 = _layer1(h0, ix_in1, ix_out1, b1)
    return h1[:N_VARS]


# speedup vs baseline: 1.5304x; 1.1740x over previous
import functools

import jax
import jax.numpy as jnp
from jax import lax
from jax.experimental import pallas as pl
from jax.experimental.pallas import tpu as pltpu
from jax.experimental.pallas import tpu_sc as plsc

N_VARS = 50000
NPAD = 50048
E_EDGES = 1600000
NW = 32
SEG_PER_TILE = 1568
SP = SEG_PER_TILE * NW
CHUNK = 4096
EDGE_PAD = 2 * CHUNK + 16
NB = 48

_MESH = plsc.VectorSubcoreMesh(
    core_axis_name="c", subcore_axis_name="s", num_cores=2, num_subcores=16
)


_GDN = lax.GatherDimensionNumbers(
    offset_dims=(), collapsed_slice_dims=(0,), start_index_map=(0,)
)


def _vgather(x, idx):
    return lax.gather(x, idx[:, None], _GDN, slice_sizes=(1,),
                      mode=lax.GatherScatterMode.PROMISE_IN_BOUNDS)


def _seg_reduce_body(tab, ixin, ixout, bounds, out, tab_v, acc,
                     bi0, bo0, bi1, bo1, bnd_v, sem_t, sem0, sem1,
                     *, transform, apply_exp):
    wid = lax.axis_index("s") * 2 + lax.axis_index("c")
    tcopy = pltpu.async_copy(tab, tab_v, sem_t)
    pltpu.sync_copy(bounds, bnd_v)
    seg_base = pl.multiple_of(wid * SEG_PER_TILE, 16)
    e_lo = bnd_v[pl.ds(wid, 16)][0]
    e_hi = bnd_v[pl.ds(wid + 1, 16)][0]

    zeros16 = jnp.zeros((16,), jnp.float32)

    def zero_body(i, _):
        acc[pl.ds(i * 16, 16)] = zeros16
        return 0

    lax.fori_loop(0, SEG_PER_TILE // 16, zero_body, 0)

    iot = lax.iota(jnp.int32, 16)
    prev_l = jnp.maximum(iot - 1, 0)
    next_l = jnp.minimum(iot + 1, 15)

    base = e_lo & ~15
    nch = (e_hi - base + CHUNK - 1) // CHUNK

    def chunk_off(k):
        pos = base + k * CHUNK
        return pl.multiple_of(jnp.minimum(pos, E_EDGES - CHUNK), 16)

    def start_dma(k, bi_, bo_, sem_):
        off = chunk_off(k)
        pltpu.async_copy(ixin.at[pl.ds(off, CHUNK)], bi_, sem_)
        pltpu.async_copy(ixout.at[pl.ds(off, CHUNK)], bo_, sem_)

    def wait_dma(bi_, bo_, sem_):
        pltpu.make_async_copy(ixin.at[pl.ds(0, CHUNK)], bi_, sem_).wait()
        pltpu.make_async_copy(ixout.at[pl.ds(0, CHUNK)], bo_, sem_).wait()

    @pl.when(nch > 0)
    def _():
        start_dma(0, bi0, bo0, sem0)

    tcopy.wait()

    def compute_chunk(k, bi_, bo_):
        off = chunk_off(k)
        lo_k = jnp.maximum(e_lo, base + k * CHUNK)

        @plsc.parallel_loop(0, CHUNK, step=16, unroll=4)
        def vreg_body(j):
            io = bo_[pl.ds(j, 16)]
            ii = bi_[pl.ds(j, 16)]
            v = plsc.load_gather(tab_v, [transform(ii)])
            incl = plsc.cumsum(v)
            excl = incl - v
            pio = _vgather(io, prev_l)
            nio = _vgather(io, next_l)
            start = (iot == 0) | (io != pio)
            last = (iot == 15) | (io != nio)
            rs = plsc.cummax(jnp.where(start, iot, 0))
            part = incl - _vgather(excl, rs)
            eid = off + j + iot
            m = last & (eid >= lo_k) & (eid < e_hi)
            plsc.addupdate_scatter(acc, [io - seg_base], part, mask=m)

    def chunk_body(k, _):
        @pl.when(k % 2 == 0)
        def _():
            wait_dma(bi0, bo0, sem0)

            @pl.when(k + 1 < nch)
            def _():
                start_dma(k + 1, bi1, bo1, sem1)

            compute_chunk(k, bi0, bo0)

        @pl.when(k % 2 == 1)
        def _():
            wait_dma(bi1, bo1, sem1)

            @pl.when(k + 1 < nch)
            def _():
                start_dma(k + 1, bi0, bo0, sem0)

            compute_chunk(k, bi1, bo1)

        return 0

    lax.fori_loop(0, nch, chunk_body, 0)

    if apply_exp:
        def exp_body(i, _):
            acc[pl.ds(i * 16, 16)] = jnp.exp(acc[pl.ds(i * 16, 16)])
            return 0

        lax.fori_loop(0, SEG_PER_TILE // 16, exp_body, 0)

    pltpu.sync_copy(acc, out.at[pl.ds(seg_base, SEG_PER_TILE)])


def _transform_layer0(ii):
    j2 = ii - 2
    return (j2 >> 1) + (j2 & 1) * NPAD


def _make_seg_kernel(tab_len, transform, apply_exp):
    scratch = [
        pltpu.VMEM((tab_len,), jnp.float32),
        pltpu.VMEM((SEG_PER_TILE,), jnp.float32),
        pltpu.VMEM((CHUNK,), jnp.int32),
        pltpu.VMEM((CHUNK,), jnp.int32),
        pltpu.VMEM((CHUNK,), jnp.int32),
        pltpu.VMEM((CHUNK,), jnp.int32),
        pltpu.VMEM((NB,), jnp.int32),
        pltpu.SemaphoreType.DMA,
        pltpu.SemaphoreType.DMA,
        pltpu.SemaphoreType.DMA,
    ]

    @functools.partial(
        pl.kernel,
        out_type=jax.ShapeDtypeStruct((SP,), jnp.float32),
        mesh=_MESH,
        scratch_types=scratch,
        compiler_params=pltpu.CompilerParams(needs_layout_passes=False),
    )
    def k(tab, ixin, ixout, bounds, out, tab_v, acc, bi0, bo0, bi1, bo1,
          bnd_v, sem_t, sem0, sem1):
        _seg_reduce_body(tab, ixin, ixout, bounds, out, tab_v, acc,
                         bi0, bo0, bi1, bo1, bnd_v, sem_t, sem0, sem1,
                         transform=transform, apply_exp=apply_exp)

    return k


_layer0 = _make_seg_kernel(2 * NPAD, _transform_layer0, True)
_layer1 = _make_seg_kernel(SP, lambda ii: ii, False)


def _log_table(x_pos):
    xp = jnp.pad(x_pos, (0, NPAD - N_VARS), constant_values=0.5)
    xp = xp.reshape(NPAD // 128, 128)

    def body(x_ref, o_ref):
        x = x_ref[...]
        o_ref[0] = jnp.log(x)
        o_ref[1] = jnp.log(1.0 - x)

    out = pl.pallas_call(
        body,
        out_shape=jax.ShapeDtypeStruct((2, NPAD // 128, 128), jnp.float32),
    )(xp)
    return out.reshape(-1)


def kernel(x_pos, ix_in0, ix_out0, ix_in1, ix_out1):
    ix_in0 = ix_in0.astype(jnp.int32)
    ix_out0 = ix_out0.astype(jnp.int32)
    ix_in1 = ix_in1.astype(jnp.int32)
    ix_out1 = ix_out1.astype(jnp.int32)

    ltab = _log_table(x_pos)

    seg_starts = jnp.arange(NW + 1, dtype=jnp.int32) * SEG_PER_TILE
    b0 = jnp.pad(
        jnp.sum(ix_out0[:, None] < seg_starts[None, :], axis=0,
                dtype=jnp.int32), (0, NB - (NW + 1)))
    b1 = jnp.pad(
        jnp.sum(ix_out1[:, None] < seg_starts[None, :], axis=0,
                dtype=jnp.int32), (0, NB - (NW + 1)))

    h0 = _layer0(ltab, ix_in0, ix_out0, b0)
    h1 = _layer1(h0, ix_in1, ix_out1, b1)
    return h1[:N_VARS]


# speedup vs baseline: 1.5502x; 1.0129x over previous
import functools

import jax
import jax.numpy as jnp
from jax import lax
from jax.experimental import pallas as pl
from jax.experimental.pallas import tpu as pltpu
from jax.experimental.pallas import tpu_sc as plsc

N_VARS = 50000
NPAD = 50048
E_EDGES = 1600000
NW = 32
SEG_PER_TILE = 1568
SP = SEG_PER_TILE * NW
CHUNK = 4096
EDGE_PAD = 2 * CHUNK + 16
NB = 48

_MESH = plsc.VectorSubcoreMesh(
    core_axis_name="c", subcore_axis_name="s", num_cores=2, num_subcores=16
)


_GDN = lax.GatherDimensionNumbers(
    offset_dims=(), collapsed_slice_dims=(0,), start_index_map=(0,)
)


def _vgather(x, idx):
    return lax.gather(x, idx[:, None], _GDN, slice_sizes=(1,),
                      mode=lax.GatherScatterMode.PROMISE_IN_BOUNDS)


def _seg_reduce_body(tab, ixin, ixout, bounds, out, tab_v, acc,
                     bi0, bo0, bi1, bo1, bnd_v, sem_t, sem0, sem1,
                     *, transform, apply_exp):
    wid = lax.axis_index("s") * 2 + lax.axis_index("c")
    tcopy = pltpu.async_copy(tab, tab_v, sem_t)
    pltpu.sync_copy(bounds, bnd_v)
    seg_base = pl.multiple_of(wid * SEG_PER_TILE, 16)
    e_lo = bnd_v[pl.ds(wid, 16)][0]
    e_hi = bnd_v[pl.ds(wid + 1, 16)][0]

    zeros16 = jnp.zeros((16,), jnp.float32)

    def zero_body(i, _):
        acc[pl.ds(i * 16, 16)] = zeros16
        return 0

    lax.fori_loop(0, SEG_PER_TILE // 16, zero_body, 0)

    iot = lax.iota(jnp.int32, 16)
    prev_l = jnp.maximum(iot - 1, 0)
    next_l = jnp.minimum(iot + 1, 15)

    base = e_lo & ~15
    nch = (e_hi - base + CHUNK - 1) // CHUNK

    def chunk_off(k):
        pos = base + k * CHUNK
        return pl.multiple_of(jnp.minimum(pos, E_EDGES - CHUNK), 16)

    def start_dma(k, bi_, bo_, sem_):
        off = chunk_off(k)
        pltpu.async_copy(ixin.at[pl.ds(off, CHUNK)], bi_, sem_)
        pltpu.async_copy(ixout.at[pl.ds(off, CHUNK)], bo_, sem_)

    def wait_dma(bi_, bo_, sem_):
        pltpu.make_async_copy(ixin.at[pl.ds(0, CHUNK)], bi_, sem_).wait()
        pltpu.make_async_copy(ixout.at[pl.ds(0, CHUNK)], bo_, sem_).wait()

    @pl.when(nch > 0)
    def _():
        start_dma(0, bi0, bo0, sem0)

    tcopy.wait()

    def compute_chunk(k, bi_, bo_):
        off = chunk_off(k)
        lo_k = jnp.maximum(e_lo, base + k * CHUNK)

        @plsc.parallel_loop(0, CHUNK, step=16, unroll=4)
        def vreg_body(j):
            io = bo_[pl.ds(j, 16)]
            ii = bi_[pl.ds(j, 16)]
            v = plsc.load_gather(tab_v, [transform(ii)])
            incl = plsc.cumsum(v)
            excl = incl - v
            pio = _vgather(io, prev_l)
            nio = _vgather(io, next_l)
            start = (iot == 0) | (io != pio)
            last = (iot == 15) | (io != nio)
            rs = plsc.cummax(jnp.where(start, iot, 0))
            part = incl - _vgather(excl, rs)
            rel = (off + j + iot - lo_k).astype(jnp.uint32)
            m = last & (rel < (e_hi - lo_k).astype(jnp.uint32))
            plsc.addupdate_scatter(acc, [io - seg_base], part, mask=m)

    def chunk_body(k, _):
        @pl.when(k % 2 == 0)
        def _():
            wait_dma(bi0, bo0, sem0)

            @pl.when(k + 1 < nch)
            def _():
                start_dma(k + 1, bi1, bo1, sem1)

            compute_chunk(k, bi0, bo0)

        @pl.when(k % 2 == 1)
        def _():
            wait_dma(bi1, bo1, sem1)

            @pl.when(k + 1 < nch)
            def _():
                start_dma(k + 1, bi0, bo0, sem0)

            compute_chunk(k, bi1, bo1)

        return 0

    lax.fori_loop(0, nch, chunk_body, 0)

    if apply_exp:
        def exp_body(i, _):
            acc[pl.ds(i * 16, 16)] = jnp.exp(acc[pl.ds(i * 16, 16)])
            return 0

        lax.fori_loop(0, SEG_PER_TILE // 16, exp_body, 0)

    pltpu.sync_copy(acc, out.at[pl.ds(seg_base, SEG_PER_TILE)])


def _transform_layer0(ii):
    j2 = ii - 2
    return (j2 >> 1) + jnp.where((j2 & 1) == 1, NPAD, 0)


def _make_seg_kernel(tab_len, transform, apply_exp):
    scratch = [
        pltpu.VMEM((tab_len,), jnp.float32),
        pltpu.VMEM((SEG_PER_TILE,), jnp.float32),
        pltpu.VMEM((CHUNK,), jnp.int32),
        pltpu.VMEM((CHUNK,), jnp.int32),
        pltpu.VMEM((CHUNK,), jnp.int32),
        pltpu.VMEM((CHUNK,), jnp.int32),
        pltpu.VMEM((NB,), jnp.int32),
        pltpu.SemaphoreType.DMA,
        pltpu.SemaphoreType.DMA,
        pltpu.SemaphoreType.DMA,
    ]

    @functools.partial(
        pl.kernel,
        out_type=jax.ShapeDtypeStruct((SP,), jnp.float32),
        mesh=_MESH,
        scratch_types=scratch,
        compiler_params=pltpu.CompilerParams(needs_layout_passes=False),
    )
    def k(tab, ixin, ixout, bounds, out, tab_v, acc, bi0, bo0, bi1, bo1,
          bnd_v, sem_t, sem0, sem1):
        _seg_reduce_body(tab, ixin, ixout, bounds, out, tab_v, acc,
                         bi0, bo0, bi1, bo1, bnd_v, sem_t, sem0, sem1,
                         transform=transform, apply_exp=apply_exp)

    return k


_layer0 = _make_seg_kernel(2 * NPAD, _transform_layer0, True)
_layer1 = _make_seg_kernel(SP, lambda ii: ii, False)


def _log_table(x_pos):
    xp = jnp.pad(x_pos, (0, NPAD - N_VARS), constant_values=0.5)
    xp = xp.reshape(NPAD // 128, 128)

    def body(x_ref, o_ref):
        x = x_ref[...]
        o_ref[0] = jnp.log(x)
        o_ref[1] = jnp.log(1.0 - x)

    out = pl.pallas_call(
        body,
        out_shape=jax.ShapeDtypeStruct((2, NPAD // 128, 128), jnp.float32),
    )(xp)
    return out.reshape(-1)


def kernel(x_pos, ix_in0, ix_out0, ix_in1, ix_out1):
    ix_in0 = ix_in0.astype(jnp.int32)
    ix_out0 = ix_out0.astype(jnp.int32)
    ix_in1 = ix_in1.astype(jnp.int32)
    ix_out1 = ix_out1.astype(jnp.int32)

    ltab = _log_table(x_pos)

    seg_starts = jnp.arange(NW + 1, dtype=jnp.int32) * SEG_PER_TILE
    b0 = jnp.pad(
        jnp.sum(ix_out0[:, None] < seg_starts[None, :], axis=0,
                dtype=jnp.int32), (0, NB - (NW + 1)))
    b1 = jnp.pad(
        jnp.sum(ix_out1[:, None] < seg_starts[None, :], axis=0,
                dtype=jnp.int32), (0, NB - (NW + 1)))

    h0 = _layer0(ltab, ix_in0, ix_out0, b0)
    h1 = _layer1(h0, ix_in1, ix_out1, b1)
    return h1[:N_VARS]


# speedup vs baseline: 2.3977x; 1.5467x over previous
import functools

import jax
import jax.numpy as jnp
from jax import lax
from jax.experimental import pallas as pl
from jax.experimental.pallas import tpu as pltpu
from jax.experimental.pallas import tpu_sc as plsc

N_VARS = 50000
NPAD = 50048
E_EDGES = 1600000
NW = 32
SEG_PER_TILE = 1568
SP = SEG_PER_TILE * NW
CHUNK = 4096
EDGE_PAD = 2 * CHUNK + 16
NB = 48
SAMPLE = 128

_MESH = plsc.VectorSubcoreMesh(
    core_axis_name="c", subcore_axis_name="s", num_cores=2, num_subcores=16
)


_GDN = lax.GatherDimensionNumbers(
    offset_dims=(), collapsed_slice_dims=(0,), start_index_map=(0,)
)


def _vgather(x, idx):
    return lax.gather(x, idx[:, None], _GDN, slice_sizes=(1,),
                      mode=lax.GatherScatterMode.PROMISE_IN_BOUNDS)


def _seg_reduce_body(tab, ixin, ixout, bounds, out, tab_v, acc,
                     bi0, bo0, bi1, bo1, bnd_v, win_v, sem_t, sem0, sem1,
                     *, transform, apply_exp):
    wid = lax.axis_index("s") * 2 + lax.axis_index("c")
    tcopy = pltpu.async_copy(tab, tab_v, sem_t)
    pltpu.sync_copy(bounds, bnd_v)
    seg_base = pl.multiple_of(wid * SEG_PER_TILE, 16)

    def refine(cb, bnd):
        w0 = pl.multiple_of(jnp.maximum(cb - 1, 0) * SAMPLE, 16)
        pltpu.sync_copy(ixout.at[pl.ds(w0, SAMPLE)], win_v)

        def cnt_body(i, c):
            w = win_v[pl.ds(i * 16, 16)]
            return c + jnp.sum(jnp.where(w < bnd, 1, 0))

        return w0 + lax.fori_loop(0, SAMPLE // 16, cnt_body, 0)

    e_lo = refine(bnd_v[pl.ds(wid, 16)][0], seg_base)
    e_hi = refine(bnd_v[pl.ds(wid + 1, 16)][0], seg_base + SEG_PER_TILE)

    zeros16 = jnp.zeros((16,), jnp.float32)

    def zero_body(i, _):
        acc[pl.ds(i * 16, 16)] = zeros16
        return 0

    lax.fori_loop(0, SEG_PER_TILE // 16, zero_body, 0)

    iot = lax.iota(jnp.int32, 16)
    prev_l = jnp.maximum(iot - 1, 0)
    next_l = jnp.minimum(iot + 1, 15)

    base = e_lo & ~15
    nch = (e_hi - base + CHUNK - 1) // CHUNK

    def chunk_off(k):
        pos = base + k * CHUNK
        return pl.multiple_of(jnp.minimum(pos, E_EDGES - CHUNK), 16)

    def start_dma(k, bi_, bo_, sem_):
        off = chunk_off(k)
        pltpu.async_copy(ixin.at[pl.ds(off, CHUNK)], bi_, sem_)
        pltpu.async_copy(ixout.at[pl.ds(off, CHUNK)], bo_, sem_)

    def wait_dma(bi_, bo_, sem_):
        pltpu.make_async_copy(ixin.at[pl.ds(0, CHUNK)], bi_, sem_).wait()
        pltpu.make_async_copy(ixout.at[pl.ds(0, CHUNK)], bo_, sem_).wait()

    @pl.when(nch > 0)
    def _():
        start_dma(0, bi0, bo0, sem0)

    tcopy.wait()

    def compute_chunk(k, bi_, bo_):
        off = chunk_off(k)
        lo_k = jnp.maximum(e_lo, base + k * CHUNK)

        @plsc.parallel_loop(0, CHUNK, step=16, unroll=4)
        def vreg_body(j):
            io = bo_[pl.ds(j, 16)]
            ii = bi_[pl.ds(j, 16)]
            v = plsc.load_gather(tab_v, [transform(ii)])
            incl = plsc.cumsum(v)
            excl = incl - v
            pio = _vgather(io, prev_l)
            nio = _vgather(io, next_l)
            start = (iot == 0) | (io != pio)
            last = (iot == 15) | (io != nio)
            rs = plsc.cummax(jnp.where(start, iot, 0))
            part = incl - _vgather(excl, rs)
            rel = (off + j + iot - lo_k).astype(jnp.uint32)
            m = last & (rel < (e_hi - lo_k).astype(jnp.uint32))
            plsc.addupdate_scatter(acc, [io - seg_base], part, mask=m)

    def chunk_body(k, _):
        @pl.when(k % 2 == 0)
        def _():
            wait_dma(bi0, bo0, sem0)

            @pl.when(k + 1 < nch)
            def _():
                start_dma(k + 1, bi1, bo1, sem1)

            compute_chunk(k, bi0, bo0)

        @pl.when(k % 2 == 1)
        def _():
            wait_dma(bi1, bo1, sem1)

            @pl.when(k + 1 < nch)
            def _():
                start_dma(k + 1, bi0, bo0, sem0)

            compute_chunk(k, bi1, bo1)

        return 0

    lax.fori_loop(0, nch, chunk_body, 0)

    if apply_exp:
        def exp_body(i, _):
            acc[pl.ds(i * 16, 16)] = jnp.exp(acc[pl.ds(i * 16, 16)])
            return 0

        lax.fori_loop(0, SEG_PER_TILE // 16, exp_body, 0)

    pltpu.sync_copy(acc, out.at[pl.ds(seg_base, SEG_PER_TILE)])


def _transform_layer0(ii):
    j2 = ii - 2
    return (j2 >> 1) + jnp.where((j2 & 1) == 1, NPAD, 0)


def _make_seg_kernel(tab_len, transform, apply_exp):
    scratch = [
        pltpu.VMEM((tab_len,), jnp.float32),
        pltpu.VMEM((SEG_PER_TILE,), jnp.float32),
        pltpu.VMEM((CHUNK,), jnp.int32),
        pltpu.VMEM((CHUNK,), jnp.int32),
        pltpu.VMEM((CHUNK,), jnp.int32),
        pltpu.VMEM((CHUNK,), jnp.int32),
        pltpu.VMEM((NB,), jnp.int32),
        pltpu.VMEM((SAMPLE,), jnp.int32),
        pltpu.SemaphoreType.DMA,
        pltpu.SemaphoreType.DMA,
        pltpu.SemaphoreType.DMA,
    ]

    @functools.partial(
        pl.kernel,
        out_type=jax.ShapeDtypeStruct((SP,), jnp.float32),
        mesh=_MESH,
        scratch_types=scratch,
        compiler_params=pltpu.CompilerParams(needs_layout_passes=False),
    )
    def k(tab, ixin, ixout, bounds, out, tab_v, acc, bi0, bo0, bi1, bo1,
          bnd_v, win_v, sem_t, sem0, sem1):
        _seg_reduce_body(tab, ixin, ixout, bounds, out, tab_v, acc,
                         bi0, bo0, bi1, bo1, bnd_v, win_v, sem_t, sem0, sem1,
                         transform=transform, apply_exp=apply_exp)

    return k


_layer0 = _make_seg_kernel(2 * NPAD, _transform_layer0, True)
_layer1 = _make_seg_kernel(SP, lambda ii: ii, False)


def _log_table(x_pos):
    xp = jnp.pad(x_pos, (0, NPAD - N_VARS), constant_values=0.5)
    xp = xp.reshape(NPAD // 128, 128)

    def body(x_ref, o_ref):
        x = x_ref[...]
        o_ref[0] = jnp.log(x)
        o_ref[1] = jnp.log(1.0 - x)

    out = pl.pallas_call(
        body,
        out_shape=jax.ShapeDtypeStruct((2, NPAD // 128, 128), jnp.float32),
    )(xp)
    return out.reshape(-1)


def kernel(x_pos, ix_in0, ix_out0, ix_in1, ix_out1):
    ix_in0 = ix_in0.astype(jnp.int32)
    ix_out0 = ix_out0.astype(jnp.int32)
    ix_in1 = ix_in1.astype(jnp.int32)
    ix_out1 = ix_out1.astype(jnp.int32)

    ltab = _log_table(x_pos)

    seg_starts = jnp.arange(NW + 1, dtype=jnp.int32) * SEG_PER_TILE
    s0 = ix_out0[::SAMPLE]
    s1 = ix_out1[::SAMPLE]
    b0 = jnp.pad(
        jnp.sum(s0[:, None] < seg_starts[None, :], axis=0,
                dtype=jnp.int32), (0, NB - (NW + 1)))
    b1 = jnp.pad(
        jnp.sum(s1[:, None] < seg_starts[None, :], axis=0,
                dtype=jnp.int32), (0, NB - (NW + 1)))

    h0 = _layer0(ltab, ix_in0, ix_out0, b0)
    h1 = _layer1(h0, ix_in1, ix_out1, b1)
    return h1[:N_VARS]


# speedup vs baseline: 2.4088x; 1.0046x over previous
import functools

import jax
import jax.numpy as jnp
from jax import lax
from jax.experimental import pallas as pl
from jax.experimental.pallas import tpu as pltpu
from jax.experimental.pallas import tpu_sc as plsc

N_VARS = 50000
NPAD = 50048
E_EDGES = 1600000
NW = 32
SEG_PER_TILE = 1568
SP = SEG_PER_TILE * NW
CHUNK = 4096
EDGE_PAD = 2 * CHUNK + 16
NB = 48
SAMPLE = 128

_MESH = plsc.VectorSubcoreMesh(
    core_axis_name="c", subcore_axis_name="s", num_cores=2, num_subcores=16
)


def _seg_reduce_body(tab, ixin, ixout, bounds, out, tab_v, acc,
                     bi0, bo0, bi1, bo1, bnd_v, win_v, sem_t, sem0, sem1,
                     *, transform, apply_exp):
    wid = lax.axis_index("s") * 2 + lax.axis_index("c")
    tcopy = pltpu.async_copy(tab, tab_v, sem_t)
    pltpu.sync_copy(bounds, bnd_v)
    seg_base = pl.multiple_of(wid * SEG_PER_TILE, 16)

    def refine(cb, bnd):
        w0 = pl.multiple_of(jnp.maximum(cb - 1, 0) * SAMPLE, 16)
        pltpu.sync_copy(ixout.at[pl.ds(w0, SAMPLE)], win_v)

        def cnt_body(i, c):
            w = win_v[pl.ds(i * 16, 16)]
            return c + jnp.sum(jnp.where(w < bnd, 1, 0))

        return w0 + lax.fori_loop(0, SAMPLE // 16, cnt_body, 0)

    e_lo = refine(bnd_v[pl.ds(wid, 16)][0], seg_base)
    e_hi = refine(bnd_v[pl.ds(wid + 1, 16)][0], seg_base + SEG_PER_TILE)

    zeros16 = jnp.zeros((16,), jnp.float32)

    def zero_body(i, _):
        acc[pl.ds(i * 16, 16)] = zeros16
        return 0

    lax.fori_loop(0, SEG_PER_TILE // 16, zero_body, 0)

    iot = lax.iota(jnp.int32, 16)
    is15 = iot == 15
    lt15 = iot < 15

    base = e_lo & ~15
    nch = (e_hi - base + CHUNK - 1) // CHUNK

    def chunk_off(k):
        pos = base + k * CHUNK
        return pl.multiple_of(jnp.minimum(pos, E_EDGES - CHUNK), 16)

    def start_dma(k, bi_, bo_, sem_):
        off = chunk_off(k)
        pltpu.async_copy(ixin.at[pl.ds(off, CHUNK)], bi_, sem_)
        pltpu.async_copy(ixout.at[pl.ds(off, CHUNK)],
                         bo_.at[pl.ds(0, CHUNK)], sem_)

    def wait_dma(bi_, bo_, sem_):
        pltpu.make_async_copy(ixin.at[pl.ds(0, CHUNK)], bi_, sem_).wait()
        pltpu.make_async_copy(ixout.at[pl.ds(0, CHUNK)],
                              bo_.at[pl.ds(0, CHUNK)], sem_).wait()

    @pl.when(nch > 0)
    def _():
        start_dma(0, bi0, bo0, sem0)

    tcopy.wait()

    def compute_chunk(k, bi_, bo_):
        off = chunk_off(k)
        lo_k = jnp.maximum(e_lo, base + k * CHUNK)
        span = (e_hi - lo_k).astype(jnp.uint32)

        @plsc.parallel_loop(0, CHUNK, step=16, unroll=4)
        def vreg_body(j):
            io = bo_[pl.ds(j, 16)]
            nio = bo_[pl.ds(j + 1, 16)]
            ii = bi_[pl.ds(j, 16)]
            v = plsc.load_gather(tab_v, [transform(ii)])
            incl = plsc.cumsum(v)
            neq = io != nio
            rel = ((off + j - lo_k) + iot).astype(jnp.uint32)
            m_add = (neq | is15) & (rel < span)
            m_sub = neq & lt15 & ((rel + 1) < span)
            plsc.addupdate_scatter(acc, [io - seg_base], incl, mask=m_add)
            plsc.addupdate_scatter(acc, [nio - seg_base], -incl, mask=m_sub)

    def chunk_body(k, _):
        @pl.when(k % 2 == 0)
        def _():
            wait_dma(bi0, bo0, sem0)

            @pl.when(k + 1 < nch)
            def _():
                start_dma(k + 1, bi1, bo1, sem1)

            compute_chunk(k, bi0, bo0)

        @pl.when(k % 2 == 1)
        def _():
            wait_dma(bi1, bo1, sem1)

            @pl.when(k + 1 < nch)
            def _():
                start_dma(k + 1, bi0, bo0, sem0)

            compute_chunk(k, bi1, bo1)

        return 0

    lax.fori_loop(0, nch, chunk_body, 0)

    if apply_exp:
        def exp_body(i, _):
            acc[pl.ds(i * 16, 16)] = jnp.exp(acc[pl.ds(i * 16, 16)])
            return 0

        lax.fori_loop(0, SEG_PER_TILE // 16, exp_body, 0)

    pltpu.sync_copy(acc, out.at[pl.ds(seg_base, SEG_PER_TILE)])


def _transform_layer0(ii):
    j2 = ii - 2
    return (j2 >> 1) + jnp.where((j2 & 1) == 1, NPAD, 0)


def _make_seg_kernel(tab_len, transform, apply_exp):
    scratch = [
        pltpu.VMEM((tab_len,), jnp.float32),
        pltpu.VMEM((SEG_PER_TILE,), jnp.float32),
        pltpu.VMEM((CHUNK,), jnp.int32),
        pltpu.VMEM((CHUNK + 16,), jnp.int32),
        pltpu.VMEM((CHUNK,), jnp.int32),
        pltpu.VMEM((CHUNK + 16,), jnp.int32),
        pltpu.VMEM((NB,), jnp.int32),
        pltpu.VMEM((SAMPLE,), jnp.int32),
        pltpu.SemaphoreType.DMA,
        pltpu.SemaphoreType.DMA,
        pltpu.SemaphoreType.DMA,
    ]

    @functools.partial(
        pl.kernel,
        out_type=jax.ShapeDtypeStruct((SP,), jnp.float32),
        mesh=_MESH,
        scratch_types=scratch,
        compiler_params=pltpu.CompilerParams(needs_layout_passes=False),
    )
    def k(tab, ixin, ixout, bounds, out, tab_v, acc, bi0, bo0, bi1, bo1,
          bnd_v, win_v, sem_t, sem0, sem1):
        _seg_reduce_body(tab, ixin, ixout, bounds, out, tab_v, acc,
                         bi0, bo0, bi1, bo1, bnd_v, win_v, sem_t, sem0, sem1,
                         transform=transform, apply_exp=apply_exp)

    return k


_layer0 = _make_seg_kernel(2 * NPAD, _transform_layer0, True)
_layer1 = _make_seg_kernel(SP, lambda ii: ii, False)


def _log_table(x_pos):
    xp = jnp.pad(x_pos, (0, NPAD - N_VARS), constant_values=0.5)
    xp = xp.reshape(NPAD // 128, 128)

    def body(x_ref, o_ref):
        x = x_ref[...]
        o_ref[0] = jnp.log(x)
        o_ref[1] = jnp.log(1.0 - x)

    out = pl.pallas_call(
        body,
        out_shape=jax.ShapeDtypeStruct((2, NPAD // 128, 128), jnp.float32),
    )(xp)
    return out.reshape(-1)


def kernel(x_pos, ix_in0, ix_out0, ix_in1, ix_out1):
    ix_in0 = ix_in0.astype(jnp.int32)
    ix_out0 = ix_out0.astype(jnp.int32)
    ix_in1 = ix_in1.astype(jnp.int32)
    ix_out1 = ix_out1.astype(jnp.int32)

    ltab = _log_table(x_pos)

    seg_starts = jnp.arange(NW + 1, dtype=jnp.int32) * SEG_PER_TILE
    s0 = ix_out0[::SAMPLE]
    s1 = ix_out1[::SAMPLE]
    b0 = jnp.pad(
        jnp.sum(s0[:, None] < seg_starts[None, :], axis=0,
                dtype=jnp.int32), (0, NB - (NW + 1)))
    b1 = jnp.pad(
        jnp.sum(s1[:, None] < seg_starts[None, :], axis=0,
                dtype=jnp.int32), (0, NB - (NW + 1)))

    h0 = _layer0(ltab, ix_in0, ix_out0, b0)
    h1 = _layer1(h0, ix_in1, ix_out1, b1)
    return h1[:N_VARS]
